# baseline (speedup 1.0000x reference)
import jax
import jax.numpy as jnp
import numpy as np
from jax import lax
from jax.experimental import pallas as pl
from jax.experimental.pallas import tpu as pltpu
from jax.experimental.pallas import tpu_sc as plsc

VOCAB = 30522
EMBED = 128
HIDDEN = 256
NUM_CLASSES = 16
B, L = 4096, 200

NC, NS, LANES = 2, 16, 16
NW = NC * NS
NSPLIT = 2
BSUB = B // NSPLIT
B_PER_W = BSUB // NW
VPR = EMBED // LANES
CHUNKS = ((0, 200),)
RED_UNROLL = 8
NBUF = 5


def _pool_body(x_hbm, emb_hbm, out_hbm, idx_v, bufs, outb_v, sems):
    wid = lax.axis_index("s") * NC + lax.axis_index("c")
    base = wid * B_PER_W

    pltpu.sync_copy(x_hbm.at[pl.ds(base, B_PER_W)], idx_v)

    def fire(rowbuf, sem, r):
        for off, n in CHUNKS:
            pltpu.async_copy(
                emb_hbm.at[idx_v.at[r, pl.ds(off, n)]],
                rowbuf.at[pl.ds(off, n)],
                sem,
            )

    def drain(rowbuf, sem, r):
        for off, n in CHUNKS:
            pltpu.make_async_copy(
                emb_hbm.at[idx_v.at[r, pl.ds(off, n)]],
                rowbuf.at[pl.ds(off, n)],
                sem,
            ).wait()

    def reduce(rowbuf, r):
        def red(k, accs):
            new = list(accs)
            for u in range(RED_UNROLL):
                row = k * RED_UNROLL + u
                for c in range(VPR // 2):
                    w = rowbuf[row, pl.ds(c * LANES, LANES)]
                    a = lax.bitcast_convert_type(w << 16, jnp.float32)
                    b = lax.bitcast_convert_type(
                        w & jnp.int32(-65536), jnp.float32
                    )
                    new[c] = new[c] + a
                    new[VPR // 2 + c] = new[VPR // 2 + c] + b
            return tuple(new)

        accs = lax.fori_loop(
            0,
            L // RED_UNROLL,
            red,
            tuple(jnp.zeros((LANES,), jnp.float32) for _ in range(VPR)),
        )
        for c in range(VPR):
            outb_v[r, pl.ds(c * LANES, LANES)] = accs[c] * (1.0 / L)

    for s in range(NBUF):
        fire(bufs[s], sems[s], s)

    NFULL = B_PER_W // NBUF

    def body(i, _):
        r0 = NBUF * i
        for s in range(NBUF):
            drain(bufs[s], sems[s], r0 + s)
            reduce(bufs[s], r0 + s)

            @pl.when(r0 + s + NBUF < B_PER_W)
            def _():
                fire(bufs[s], sems[s], r0 + s + NBUF)

        return 0

    lax.fori_loop(0, NFULL, body, 0)
    for r in range(NFULL * NBUF, B_PER_W):
        drain(bufs[r % NBUF], sems[r % NBUF], r)
        reduce(bufs[r % NBUF], r)
    pltpu.sync_copy(outb_v, out_hbm.at[pl.ds(base, B_PER_W)])


def _pool_entry(x_hbm, emb_hbm, out_hbm, idx_v, b0, b1, b2, b3, b4, outb_v,
                s0, s1, s2, s3, s4):
    _pool_body(x_hbm, emb_hbm, out_hbm, idx_v, (b0, b1, b2, b3, b4), outb_v,
               (s0, s1, s2, s3, s4))


def _sc_pool(x_flat, emb16):
    mesh = plsc.VectorSubcoreMesh(
        core_axis_name="c", subcore_axis_name="s", num_cores=NC, num_subcores=NS
    )
    return pl.kernel(
        _pool_entry,
        out_type=jax.ShapeDtypeStruct((BSUB, EMBED), jnp.float32),
        mesh=mesh,
        compiler_params=pltpu.CompilerParams(
            use_tc_tiling_on_sc=False, needs_layout_passes=False
        ),
        scratch_types=[
            pltpu.VMEM((B_PER_W, L), jnp.int32),
            pltpu.VMEM((L, EMBED // 2), jnp.int32),
            pltpu.VMEM((L, EMBED // 2), jnp.int32),
            pltpu.VMEM((L, EMBED // 2), jnp.int32),
            pltpu.VMEM((L, EMBED // 2), jnp.int32),
            pltpu.VMEM((L, EMBED // 2), jnp.int32),
            pltpu.VMEM((B_PER_W, EMBED), jnp.float32),
            pltpu.SemaphoreType.DMA,
            pltpu.SemaphoreType.DMA,
            pltpu.SemaphoreType.DMA,
            pltpu.SemaphoreType.DMA,
            pltpu.SemaphoreType.DMA,
        ],
    )(x_flat, emb16)


def _pack_body(e_ref, o_ref):
    b16 = lax.bitcast_convert_type(
        e_ref[...].astype(jnp.bfloat16), jnp.uint16
    )
    lo = b16[:, : EMBED // 2].astype(jnp.uint32)
    hi = b16[:, EMBED // 2 :].astype(jnp.uint32)
    o_ref[...] = lax.bitcast_convert_type(lo | (hi << 16), jnp.int32)


def _tc_pack(emb):
    blk = 1024
    grid = pl.cdiv(VOCAB, blk)
    return pl.pallas_call(
        _pack_body,
        grid=(grid,),
        in_specs=[pl.BlockSpec((blk, EMBED), lambda i: (i, 0))],
        out_specs=pl.BlockSpec((blk, EMBED // 2), lambda i: (i, 0)),
        out_shape=jax.ShapeDtypeStruct((VOCAB, EMBED // 2), jnp.int32),
    )(emb)


def _mlp_body(p_ref, w1_ref, b1_ref, w2_ref, b2_ref, o_ref):
    h = jnp.dot(p_ref[...], w1_ref[...], preferred_element_type=jnp.float32)
    h = jnp.maximum(h + b1_ref[...], 0.0)
    o_ref[...] = (
        jnp.dot(h, w2_ref[...], preferred_element_type=jnp.float32) + b2_ref[...]
    )


def _tc_mlp(pooled, W1, b1, W2, b2):
    blk = 512
    grid = BSUB // blk
    return pl.pallas_call(
        _mlp_body,
        grid=(grid,),
        in_specs=[
            pl.BlockSpec((blk, EMBED), lambda i: (i, 0)),
            pl.BlockSpec((EMBED, HIDDEN), lambda i: (0, 0)),
            pl.BlockSpec((1, HIDDEN), lambda i: (0, 0)),
            pl.BlockSpec((HIDDEN, NUM_CLASSES), lambda i: (0, 0)),
            pl.BlockSpec((1, NUM_CLASSES), lambda i: (0, 0)),
        ],
        out_specs=pl.BlockSpec((blk, NUM_CLASSES), lambda i: (i, 0)),
        out_shape=jax.ShapeDtypeStruct((BSUB, NUM_CLASSES), jnp.float32),
    )(pooled, W1, b1.reshape(1, HIDDEN), W2, b2.reshape(1, NUM_CLASSES))


def kernel(x, emb, W1, b1, W2, b2):
    emb16 = _tc_pack(emb)
    xi = x.astype(jnp.int32)
    outs = []
    for h in range(NSPLIT):
        pooled = _sc_pool(xi[h * BSUB:(h + 1) * BSUB], emb16)
        outs.append(_tc_mlp(pooled, W1, b1, W2, b2))
    return jnp.concatenate(outs, axis=0)


# speedup vs baseline: 1.0167x; 1.0167x over previous
import jax
import jax.numpy as jnp
import numpy as np
from jax import lax
from jax.experimental import pallas as pl
from jax.experimental.pallas import tpu as pltpu
from jax.experimental.pallas import tpu_sc as plsc

VOCAB = 30522
EMBED = 128
HIDDEN = 256
NUM_CLASSES = 16
B, L = 4096, 200

NC, NS, LANES = 2, 16, 16
NW = NC * NS
B_PER_W = B // NW
VPR = EMBED // LANES
CHUNKS = ((0, 200),)
RED_UNROLL = 8
NBUF = 5


def _pool_body(x_hbm, emb_hbm, out_hbm, idx_v, bufs, outb_v, sems):
    wid = lax.axis_index("s") * NC + lax.axis_index("c")
    base = wid * B_PER_W

    pltpu.sync_copy(x_hbm.at[pl.ds(base, B_PER_W)], idx_v)

    def fire(rowbuf, sem, r):
        for off, n in CHUNKS:
            pltpu.async_copy(
                emb_hbm.at[idx_v.at[r, pl.ds(off, n)]],
                rowbuf.at[pl.ds(off, n)],
                sem,
            )

    def drain(rowbuf, sem, r):
        for off, n in CHUNKS:
            pltpu.make_async_copy(
                emb_hbm.at[idx_v.at[r, pl.ds(off, n)]],
                rowbuf.at[pl.ds(off, n)],
                sem,
            ).wait()

    def reduce(rowbuf, r):
        def red(k, accs):
            new = list(accs)
            for u in range(RED_UNROLL):
                row = k * RED_UNROLL + u
                for c in range(VPR // 2):
                    w = rowbuf[row, pl.ds(c * LANES, LANES)]
                    a = lax.bitcast_convert_type(w << 16, jnp.float32)
                    b = lax.bitcast_convert_type(
                        w & jnp.int32(-65536), jnp.float32
                    )
                    new[c] = new[c] + a
                    new[VPR // 2 + c] = new[VPR // 2 + c] + b
            return tuple(new)

        accs = lax.fori_loop(
            0,
            L // RED_UNROLL,
            red,
            tuple(jnp.zeros((LANES,), jnp.float32) for _ in range(VPR)),
        )
        for c in range(VPR):
            outb_v[r, pl.ds(c * LANES, LANES)] = accs[c] * (1.0 / L)

    for s in range(NBUF):
        fire(bufs[s], sems[s], s)

    NFULL = B_PER_W // NBUF

    def body(i, _):
        r0 = NBUF * i
        for s in range(NBUF):
            drain(bufs[s], sems[s], r0 + s)
            reduce(bufs[s], r0 + s)

            @pl.when(r0 + s + NBUF < B_PER_W)
            def _():
                fire(bufs[s], sems[s], r0 + s + NBUF)

        return 0

    lax.fori_loop(0, NFULL, body, 0)
    for r in range(NFULL * NBUF, B_PER_W):
        drain(bufs[r % NBUF], sems[r % NBUF], r)
        reduce(bufs[r % NBUF], r)
    pltpu.sync_copy(outb_v, out_hbm.at[pl.ds(base, B_PER_W)])


def _pool_entry(x_hbm, emb_hbm, out_hbm, idx_v, b0, b1, b2, b3, b4, outb_v,
                s0, s1, s2, s3, s4):
    _pool_body(x_hbm, emb_hbm, out_hbm, idx_v, (b0, b1, b2, b3, b4), outb_v,
               (s0, s1, s2, s3, s4))


def _sc_pool(x_flat, emb16):
    mesh = plsc.VectorSubcoreMesh(
        core_axis_name="c", subcore_axis_name="s", num_cores=NC, num_subcores=NS
    )
    return pl.kernel(
        _pool_entry,
        out_type=jax.ShapeDtypeStruct((B, EMBED), jnp.float32),
        mesh=mesh,
        compiler_params=pltpu.CompilerParams(
            use_tc_tiling_on_sc=False, needs_layout_passes=False
        ),
        scratch_types=[
            pltpu.VMEM((B_PER_W, L), jnp.int32),
            pltpu.VMEM((L, EMBED // 2), jnp.int32),
            pltpu.VMEM((L, EMBED // 2), jnp.int32),
            pltpu.VMEM((L, EMBED // 2), jnp.int32),
            pltpu.VMEM((L, EMBED // 2), jnp.int32),
            pltpu.VMEM((L, EMBED // 2), jnp.int32),
            pltpu.VMEM((B_PER_W, EMBED), jnp.float32),
            pltpu.SemaphoreType.DMA,
            pltpu.SemaphoreType.DMA,
            pltpu.SemaphoreType.DMA,
            pltpu.SemaphoreType.DMA,
            pltpu.SemaphoreType.DMA,
        ],
    )(x_flat, emb16)


def _pack_body(e_ref, o_ref):
    b16 = lax.bitcast_convert_type(
        e_ref[...].astype(jnp.bfloat16), jnp.uint16
    )
    lo = b16[:, : EMBED // 2].astype(jnp.uint32)
    hi = b16[:, EMBED // 2 :].astype(jnp.uint32)
    o_ref[...] = lax.bitcast_convert_type(lo | (hi << 16), jnp.int32)


def _tc_pack(emb):
    blk = 1024
    grid = pl.cdiv(VOCAB, blk)
    return pl.pallas_call(
        _pack_body,
        grid=(grid,),
        in_specs=[pl.BlockSpec((blk, EMBED), lambda i: (i, 0))],
        out_specs=pl.BlockSpec((blk, EMBED // 2), lambda i: (i, 0)),
        out_shape=jax.ShapeDtypeStruct((VOCAB, EMBED // 2), jnp.int32),
    )(emb)


def _mlp_body(p_ref, w1_ref, b1_ref, w2_ref, b2_ref, o_ref):
    h = jnp.dot(p_ref[...], w1_ref[...], preferred_element_type=jnp.float32)
    h = jnp.maximum(h + b1_ref[...], 0.0)
    o_ref[...] = (
        jnp.dot(h, w2_ref[...], preferred_element_type=jnp.float32) + b2_ref[...]
    )


def _tc_mlp(pooled, W1, b1, W2, b2):
    blk = 512
    grid = B // blk
    return pl.pallas_call(
        _mlp_body,
        grid=(grid,),
        in_specs=[
            pl.BlockSpec((blk, EMBED), lambda i: (i, 0)),
            pl.BlockSpec((EMBED, HIDDEN), lambda i: (0, 0)),
            pl.BlockSpec((1, HIDDEN), lambda i: (0, 0)),
            pl.BlockSpec((HIDDEN, NUM_CLASSES), lambda i: (0, 0)),
            pl.BlockSpec((1, NUM_CLASSES), lambda i: (0, 0)),
        ],
        out_specs=pl.BlockSpec((blk, NUM_CLASSES), lambda i: (i, 0)),
        out_shape=jax.ShapeDtypeStruct((B, NUM_CLASSES), jnp.float32),
    )(pooled, W1, b1.reshape(1, HIDDEN), W2, b2.reshape(1, NUM_CLASSES))


def kernel(x, emb, W1, b1, W2, b2):
    emb16 = _tc_pack(emb)
    pooled = _sc_pool(x.astype(jnp.int32), emb16)
    return _tc_mlp(pooled, W1, b1, W2, b2)
